# Initial kernel scaffold; baseline (speedup 1.0000x reference)
import jax
import jax.numpy as jnp
from jax.experimental import pallas as pl


def kernel(x, pose_adj, W, b):
    raise NotImplementedError("write your pallas kernel here")


# speedup vs baseline: 5.8973x; 5.8973x over previous
import functools

import jax
import jax.numpy as jnp
from jax.experimental import pallas as pl

TOPK = 4
GAMMA = 0.1


def _agl_kernel(x_ref, pose_ref, w_ref, b_ref, h_ref, adj_ref, *, bb, k, din, dout):
    xb = x_ref[...]
    w = w_ref[...]
    bias = b_ref[...]

    h3 = jax.lax.dot_general(
        xb, w, (((2,), (0,)), ((), ())),
        preferred_element_type=jnp.float32,
    ) + bias[None]
    h_ref[...] = h3

    sq = jnp.sum(h3 * h3, axis=2)

    g = jax.lax.dot_general(
        h3, h3, (((2,), (2,)), ((0,), (0,))),
        preferred_element_type=jnp.float32,
    )
    d2 = sq[:, :, None] + sq[:, None, :] - 2.0 * g
    d2 = jnp.maximum(d2, 1e-12)

    col = jax.lax.broadcasted_iota(jnp.int32, (1, 1, k), 2)
    rank = jnp.zeros((bb, k, k), dtype=jnp.int32)
    for jp in range(k):
        v = d2[:, :, jp:jp + 1]
        beats = (v < d2) | ((v == d2) & (jp < col))
        rank = rank + beats.astype(jnp.int32)
    knn = (rank < TOPK).astype(jnp.float32)

    row = jax.lax.broadcasted_iota(jnp.int32, (1, k, k), 1)
    eye = (row == col).astype(jnp.float32)
    blended = (pose_ref[...] + GAMMA * (knn + eye)) / (1.0 + GAMMA)
    norm = jnp.maximum(jnp.sum(jnp.abs(blended), axis=2, keepdims=True), 1e-12)
    adj_ref[...] = blended / norm


@jax.jit
def kernel(x, pose_adj, W, b):
    B, K, DIN = x.shape
    DOUT = W.shape[1]
    BB = 256
    grid = (B // BB,)
    b2 = b.reshape(1, DOUT)

    h, adj = pl.pallas_call(
        functools.partial(_agl_kernel, bb=BB, k=K, din=DIN, dout=DOUT),
        grid=grid,
        in_specs=[
            pl.BlockSpec((BB, K, DIN), lambda i: (i, 0, 0)),
            pl.BlockSpec((BB, K, K), lambda i: (i, 0, 0)),
            pl.BlockSpec((DIN, DOUT), lambda i: (0, 0)),
            pl.BlockSpec((1, DOUT), lambda i: (0, 0)),
        ],
        out_specs=[
            pl.BlockSpec((BB, K, DOUT), lambda i: (i, 0, 0)),
            pl.BlockSpec((BB, K, K), lambda i: (i, 0, 0)),
        ],
        out_shape=[
            jax.ShapeDtypeStruct((B, K, DOUT), jnp.float32),
            jax.ShapeDtypeStruct((B, K, K), jnp.float32),
        ],
    )(x, pose_adj, W, b2)
    return (h, adj)


# speedup vs baseline: 12.6948x; 2.1526x over previous
import functools

import jax
import jax.numpy as jnp
from jax.experimental import pallas as pl

TOPK = 4
GAMMA = 0.1


def _agl_kernel(x_ref, pose_ref, w_ref, b_ref, h_ref, adj_ref, *, bb, k, din, dout):
    xb = x_ref[...]
    w = w_ref[...]
    bias = b_ref[...]

    h3 = jax.lax.dot_general(
        xb, w, (((2,), (0,)), ((), ())),
        preferred_element_type=jnp.float32,
    ) + bias[None]
    h_ref[...] = h3

    sq = jnp.sum(h3 * h3, axis=2)

    g = jax.lax.dot_general(
        h3, h3, (((2,), (2,)), ((0,), (0,))),
        preferred_element_type=jnp.float32,
    )
    d2 = sq[:, :, None] + sq[:, None, :] - 2.0 * g
    d2 = jnp.maximum(d2, 1e-12)

    work = d2
    for _ in range(TOPK - 1):
        m = jnp.min(work, axis=2, keepdims=True)
        work = jnp.where(work <= m, jnp.float32(jnp.inf), work)
    thresh = jnp.min(work, axis=2, keepdims=True)
    knn = (d2 <= thresh).astype(jnp.float32)

    col = jax.lax.broadcasted_iota(jnp.int32, (1, 1, k), 2)
    row = jax.lax.broadcasted_iota(jnp.int32, (1, k, k), 1)
    eye = (row == col).astype(jnp.float32)
    blended = (pose_ref[...] + GAMMA * (knn + eye)) / (1.0 + GAMMA)
    norm = jnp.maximum(jnp.sum(jnp.abs(blended), axis=2, keepdims=True), 1e-12)
    adj_ref[...] = blended / norm


@jax.jit
def kernel(x, pose_adj, W, b):
    B, K, DIN = x.shape
    DOUT = W.shape[1]
    BB = 256
    grid = (B // BB,)
    b2 = b.reshape(1, DOUT)

    h, adj = pl.pallas_call(
        functools.partial(_agl_kernel, bb=BB, k=K, din=DIN, dout=DOUT),
        grid=grid,
        in_specs=[
            pl.BlockSpec((BB, K, DIN), lambda i: (i, 0, 0)),
            pl.BlockSpec((BB, K, K), lambda i: (i, 0, 0)),
            pl.BlockSpec((DIN, DOUT), lambda i: (0, 0)),
            pl.BlockSpec((1, DOUT), lambda i: (0, 0)),
        ],
        out_specs=[
            pl.BlockSpec((BB, K, DOUT), lambda i: (i, 0, 0)),
            pl.BlockSpec((BB, K, K), lambda i: (i, 0, 0)),
        ],
        out_shape=[
            jax.ShapeDtypeStruct((B, K, DOUT), jnp.float32),
            jax.ShapeDtypeStruct((B, K, K), jnp.float32),
        ],
    )(x, pose_adj, W, b2)
    return (h, adj)
